# speedup vs baseline: 1.2586x; 1.2586x over previous
import jax
import jax.numpy as jnp
from jax.experimental import pallas as pl
from jax.experimental.pallas import tpu as pltpu

_TILE = 4096


def _fused_kernel(x_ref, w_ref, b_ref, out_ref):
    z = jnp.dot(x_ref[...], w_ref[...], preferred_element_type=jnp.float32)
    out_ref[...] = jnp.maximum(z + b_ref[...], 0.0)


def kernel(x, beta, w, b):
    n, d = x.shape
    e = w.shape[1]
    w_eff = beta * w
    b_row = b.reshape(1, e)

    tile = min(_TILE, ((n + 7) // 8) * 8)
    n_pad = ((n + tile - 1) // tile) * tile
    if n_pad != n:
        x = jnp.pad(x, ((0, n_pad - n), (0, 0)))

    out = pl.pallas_call(
        _fused_kernel,
        out_shape=jax.ShapeDtypeStruct((n_pad, e), jnp.float32),
        grid=(n_pad // tile,),
        in_specs=[
            pl.BlockSpec((tile, d), lambda i: (i, 0)),
            pl.BlockSpec((d, e), lambda i: (0, 0)),
            pl.BlockSpec((1, e), lambda i: (0, 0)),
        ],
        out_specs=pl.BlockSpec((tile, e), lambda i: (i, 0)),
        compiler_params=pltpu.CompilerParams(
            dimension_semantics=("parallel",)),
    )(x, w_eff, b_row)
    return out[:n]


# speedup vs baseline: 1.3342x; 1.0601x over previous
import jax
import jax.numpy as jnp
from jax.experimental import pallas as pl
from jax.experimental.pallas import tpu as pltpu

_TILE = 16384


def _fused_kernel(x_ref, w_ref, b_ref, out_ref):
    z = jnp.dot(x_ref[...], w_ref[...], preferred_element_type=jnp.float32)
    out_ref[...] = jnp.maximum(z + b_ref[...], 0.0)


def kernel(x, beta, w, b):
    n, d = x.shape
    e = w.shape[1]
    w_eff = beta * w
    b_row = b.reshape(1, e)

    tile = min(_TILE, ((n + 7) // 8) * 8)
    n_pad = ((n + tile - 1) // tile) * tile
    if n_pad != n:
        x = jnp.pad(x, ((0, n_pad - n), (0, 0)))

    out = pl.pallas_call(
        _fused_kernel,
        out_shape=jax.ShapeDtypeStruct((n_pad, e), jnp.float32),
        grid=(n_pad // tile,),
        in_specs=[
            pl.BlockSpec((tile, d), lambda i: (i, 0)),
            pl.BlockSpec((d, e), lambda i: (0, 0)),
            pl.BlockSpec((1, e), lambda i: (0, 0)),
        ],
        out_specs=pl.BlockSpec((tile, e), lambda i: (i, 0)),
        compiler_params=pltpu.CompilerParams(
            dimension_semantics=("parallel",)),
    )(x, w_eff, b_row)
    return out[:n]
